# baseline (speedup 1.0000x reference)
import functools

import jax
import jax.numpy as jnp
import numpy as np
from jax import lax
from jax.experimental import pallas as pl
from jax.experimental.pallas import tpu as pltpu
from jax.experimental.pallas import tpu_sc as plsc

_N = 10000
_E = 320000
_D = 128
_T = 16
_NC = 2
_NS = 16
_NPAD = 10240
_RPT = _NPAD // _NS
_EPC = _E // _NC
_EPT = _EPC // _NS
_CB = 2000
_NBC = _EPT // _CB
_SUB = 80
_NSUB = _CB // _SUB
_VPS = _SUB // 16


def _pre_body(x_ref, w_ref, p_ref, y_ref, s_ref):
    xb = x_ref[...]
    y_ref[...] = jnp.dot(xb, w_ref[...], preferred_element_type=jnp.float32) * 2.0
    s_ref[...] = jnp.dot(xb, p_ref[...].T, preferred_element_type=jnp.float32) * (
        -1.0 / np.sqrt(np.float32(_D)))


def _pre(x, w_msg, etype_params):
    bn = 1000
    grid = (_N // bn,)
    return pl.pallas_call(
        _pre_body,
        grid=grid,
        in_specs=[
            pl.BlockSpec((bn, _D), lambda i: (i, 0)),
            pl.BlockSpec((_D, _D), lambda i: (0, 0)),
            pl.BlockSpec((_T, _D), lambda i: (0, 0)),
        ],
        out_specs=[
            pl.BlockSpec((bn, _D), lambda i: (i, 0)),
            pl.BlockSpec((bn, _T), lambda i: (i, 0)),
        ],
        out_shape=[
            jax.ShapeDtypeStruct((_N, _D), jnp.float32),
            jax.ShapeDtypeStruct((_N, _T), jnp.float32),
        ],
    )(x, w_msg, etype_params)


def _post_body(x_ref, a0_ref, a1_ref, d0_ref, d1_ref, w_ref, o_ref):
    scale = 1.0 / (d0_ref[...] + d1_ref[...] + 1e-9)
    aggr = (a0_ref[...] + a1_ref[...]) * scale
    o_ref[...] = x_ref[...] + jnp.tanh(
        jnp.dot(aggr, w_ref[...], preferred_element_type=jnp.float32))


def _post(x, a0, a1, d0, d1, w_upd):
    bn = 1000
    grid = (_N // bn,)
    return pl.pallas_call(
        _post_body,
        grid=grid,
        in_specs=[
            pl.BlockSpec((bn, _D), lambda i: (i, 0)),
            pl.BlockSpec((bn, _D), lambda i: (i, 0)),
            pl.BlockSpec((bn, _D), lambda i: (i, 0)),
            pl.BlockSpec((bn, 1), lambda i: (i, 0)),
            pl.BlockSpec((bn, 1), lambda i: (i, 0)),
            pl.BlockSpec((_D, _D), lambda i: (0, 0)),
        ],
        out_specs=pl.BlockSpec((bn, _D), lambda i: (i, 0)),
        out_shape=jax.ShapeDtypeStruct((_N, _D), jnp.float32),
    )(x, a0, a1, d0, d1, w_upd)


_sc_mesh = plsc.VectorSubcoreMesh(core_axis_name="c", subcore_axis_name="s")


@functools.partial(
    pl.kernel,
    out_type=[jax.ShapeDtypeStruct((_NC * _NPAD, _D), jnp.float32),
              jax.ShapeDtypeStruct((_NC * _NPAD,), jnp.float32)],
    mesh=_sc_mesh,
    compiler_params=pltpu.CompilerParams(
        needs_layout_passes=False, use_tc_tiling_on_sc=False),
    scratch_types=[
        pltpu.VMEM_SHARED((_NPAD, _D), jnp.float32),
        pltpu.VMEM_SHARED((_NPAD,), jnp.float32),
        pltpu.VMEM((_T, _D), jnp.float32),
        pltpu.VMEM((_CB,), jnp.int32),
        pltpu.VMEM((_CB,), jnp.int32),
        pltpu.VMEM((_CB,), jnp.int32),
        pltpu.VMEM((_CB,), jnp.float32),
        pltpu.VMEM((_CB,), jnp.float32),
        pltpu.VMEM((_NSUB, _SUB), jnp.int32),
        pltpu.VMEM((_NSUB, _SUB), jnp.int32),
        pltpu.VMEM((_NSUB, _SUB), jnp.int32),
        pltpu.VMEM((_SUB, _D), jnp.float32),
        pltpu.VMEM((_SUB, _D), jnp.float32),
        pltpu.SemaphoreType.DMA,
        pltpu.SemaphoreType.DMA,
        pltpu.SemaphoreType.DMA,
    ],
)
def _sc_edges(y_hbm, sflat_hbm, p_hbm, src_hbm, dst_hbm, et_hbm, attn_hbm,
              z2d_hbm, zd_hbm, out_hbm, dnout_hbm,
              aggr_sh, denom_sh, p_l,
              src_b, dst_b, et_b, na_b, sv_b,
              sidx2, yidx2, dst2, rows0, rows1, semg, sems, semd):
    c = lax.axis_index("c")
    s = lax.axis_index("s")
    base = c * _EPC + s * _EPT
    col0 = lax.iota(jnp.int32, 16)

    pltpu.sync_copy(z2d_hbm, aggr_sh.at[pl.ds(s * _RPT, _RPT)])
    pltpu.sync_copy(zd_hbm, denom_sh.at[pl.ds(s * _RPT, _RPT)])
    pltpu.sync_copy(p_hbm, p_l)
    plsc.subcore_barrier()

    def _compute(rbuf, r):
        @plsc.parallel_loop(0, _SUB, 1, unroll=4)
        def _edge(e):
            psplat = jnp.full((16,), r * _SUB + e, jnp.int32)
            rsplat = plsc.load_gather(et_b, [psplat])
            na = plsc.load_gather(na_b, [psplat])
            for q in range(_D // 16):
                pv = plsc.load_gather(p_l, [rsplat, col0 + q * 16])
                yv = rbuf[e, pl.ds(q * 16, 16)]
                u = jnp.exp(yv + pv)
                rbuf[e, pl.ds(q * 16, 16)] = (1.0 - 2.0 / (u + 1.0)) * na

    def _wait_gather(rbuf):
        pltpu.make_async_copy(y_hbm.at[yidx2.at[0]], rbuf, semg).wait()

    def _drain_scatter(rbuf):
        pltpu.make_async_copy(rbuf, aggr_sh.at[dst2.at[0]], sems).wait()

    def chunk(bc, carry):
        off = base + bc * _CB
        pltpu.sync_copy(src_hbm.at[pl.ds(off, _CB)], src_b)
        pltpu.sync_copy(dst_hbm.at[pl.ds(off, _CB)], dst_b)
        pltpu.sync_copy(et_hbm.at[pl.ds(off, _CB)], et_b)
        pltpu.sync_copy(attn_hbm.at[pl.ds(off, _CB)], na_b)

        @plsc.parallel_loop(0, _NSUB, 1, unroll=2)
        def _mk(r):
            for q in range(_VPS):
                j = r * _SUB + q * 16
                sv = src_b[pl.ds(j, 16)]
                ev = et_b[pl.ds(j, 16)]
                sidx2[r, pl.ds(q * 16, 16)] = sv * _T + ev
                yidx2[r, pl.ds(q * 16, 16)] = sv
                dst2[r, pl.ds(q * 16, 16)] = dst_b[pl.ds(j, 16)]

        def fire_g(r, _):
            pltpu.async_copy(sflat_hbm.at[sidx2.at[r]],
                             sv_b.at[pl.ds(r * _SUB, _SUB)], semg)
            return _
        lax.fori_loop(0, _NSUB, fire_g, None)

        def drain_g(r, _):
            pltpu.make_async_copy(sflat_hbm.at[sidx2.at[0]],
                                  sv_b.at[pl.ds(0, _SUB)], semg).wait()
            return _
        lax.fori_loop(0, _NSUB, drain_g, None)

        @plsc.parallel_loop(0, _CB // 16, 1, unroll=4)
        def _sig(j):
            sv = sv_b[pl.ds(j * 16, 16)]
            t = 1.0 / (1.0 + jnp.exp(sv))
            na_b[pl.ds(j * 16, 16)] = na_b[pl.ds(j * 16, 16)] * t

        def fire_d(r, _):
            pltpu.async_copy(na_b.at[pl.ds(r * _SUB, _SUB)],
                             denom_sh.at[dst2.at[r]], semd, add=True)
            return _
        lax.fori_loop(0, _NSUB, fire_d, None)

        pltpu.async_copy(y_hbm.at[yidx2.at[0]], rows0, semg)

        def pair(k, _):
            r0 = 2 * k
            _wait_gather(rows0)

            @pl.when(k > 0)
            def _():
                _drain_scatter(rows1)
            pltpu.async_copy(y_hbm.at[yidx2.at[r0 + 1]], rows1, semg)
            _compute(rows0, r0)
            pltpu.async_copy(rows0, aggr_sh.at[dst2.at[r0]], sems, add=True)
            _wait_gather(rows1)
            _drain_scatter(rows0)
            pltpu.async_copy(y_hbm.at[yidx2.at[r0 + 2]], rows0, semg)
            _compute(rows1, r0 + 1)
            pltpu.async_copy(rows1, aggr_sh.at[dst2.at[r0 + 1]], sems, add=True)
            return _
        lax.fori_loop(0, _NSUB // 2, pair, None)
        _wait_gather(rows0)
        _drain_scatter(rows1)
        _compute(rows0, _NSUB - 1)
        pltpu.async_copy(rows0, aggr_sh.at[dst2.at[_NSUB - 1]], sems, add=True)
        _drain_scatter(rows0)

        def drain_d(r, _):
            pltpu.make_async_copy(na_b.at[pl.ds(0, _SUB)],
                                  denom_sh.at[dst2.at[0]], semd).wait()
            return _
        lax.fori_loop(0, _NSUB, drain_d, None)
        return carry
    lax.fori_loop(0, _NBC, chunk, None)

    plsc.subcore_barrier()
    pltpu.sync_copy(aggr_sh.at[pl.ds(s * _RPT, _RPT)],
                    out_hbm.at[pl.ds(c * _NPAD + s * _RPT, _RPT)])
    pltpu.sync_copy(denom_sh.at[pl.ds(s * _RPT, _RPT)],
                    dnout_hbm.at[pl.ds(c * _NPAD + s * _RPT, _RPT)])


def kernel(x, attn, W_msg, etype_params, W_upd, edge_index, edge_type):
    src = edge_index[0]
    dst = edge_index[1]
    y, S = _pre(x, W_msg, etype_params)
    sflat = S.reshape(_N * _T)
    pT = etype_params * 2.0
    z2d = jnp.zeros((_RPT, _D), jnp.float32)
    zd = jnp.zeros((_RPT,), jnp.float32)
    aggr_flat, dn_flat = _sc_edges(y, sflat, pT, src, dst, edge_type, attn,
                                   z2d, zd)
    a = aggr_flat.reshape(_NC, _NPAD, _D)
    dn = dn_flat.reshape(_NC, _NPAD)
    return _post(x, a[0, :_N], a[1, :_N],
                 dn[0, :_N].reshape(_N, 1), dn[1, :_N].reshape(_N, 1), W_upd)


# speedup vs baseline: 1.5789x; 1.5789x over previous
import functools

import jax
import jax.numpy as jnp
import numpy as np
from jax import lax
from jax.experimental import pallas as pl
from jax.experimental.pallas import tpu as pltpu
from jax.experimental.pallas import tpu_sc as plsc

_N = 10000
_E = 320000
_D = 128
_T = 16
_NC = 2
_NS = 16
_NPAD = 10240
_RPT = _NPAD // _NS
_EPC = _E // _NC
_EPT = _EPC // _NS
_CB = 2000
_NBC = _EPT // _CB
_SUB = 80
_NSUB = _CB // _SUB
_VPS = _SUB // 16


def _pre_body(x_ref, w_ref, p_ref, y_ref, s_ref):
    xb = x_ref[...]
    y_ref[...] = jnp.dot(xb, w_ref[...], preferred_element_type=jnp.float32) * 2.0
    s_ref[...] = jnp.dot(xb, p_ref[...].T, preferred_element_type=jnp.float32) * (
        -1.0 / np.sqrt(np.float32(_D)))


def _pre(x, w_msg, etype_params):
    bn = 1000
    grid = (_N // bn,)
    return pl.pallas_call(
        _pre_body,
        grid=grid,
        in_specs=[
            pl.BlockSpec((bn, _D), lambda i: (i, 0)),
            pl.BlockSpec((_D, _D), lambda i: (0, 0)),
            pl.BlockSpec((_T, _D), lambda i: (0, 0)),
        ],
        out_specs=[
            pl.BlockSpec((bn, _D), lambda i: (i, 0)),
            pl.BlockSpec((bn, _T), lambda i: (i, 0)),
        ],
        out_shape=[
            jax.ShapeDtypeStruct((_N, _D), jnp.float32),
            jax.ShapeDtypeStruct((_N, _T), jnp.float32),
        ],
    )(x, w_msg, etype_params)


def _post_body(x_ref, a0_ref, a1_ref, d0_ref, d1_ref, w_ref, o_ref):
    scale = 1.0 / (d0_ref[...] + d1_ref[...] + 1e-9)
    aggr = (a0_ref[...] + a1_ref[...]) * scale
    o_ref[...] = x_ref[...] + jnp.tanh(
        jnp.dot(aggr, w_ref[...], preferred_element_type=jnp.float32))


def _post(x, a0, a1, d0, d1, w_upd):
    bn = 1000
    grid = (_N // bn,)
    return pl.pallas_call(
        _post_body,
        grid=grid,
        in_specs=[
            pl.BlockSpec((bn, _D), lambda i: (i, 0)),
            pl.BlockSpec((bn, _D), lambda i: (i, 0)),
            pl.BlockSpec((bn, _D), lambda i: (i, 0)),
            pl.BlockSpec((bn, 1), lambda i: (i, 0)),
            pl.BlockSpec((bn, 1), lambda i: (i, 0)),
            pl.BlockSpec((_D, _D), lambda i: (0, 0)),
        ],
        out_specs=pl.BlockSpec((bn, _D), lambda i: (i, 0)),
        out_shape=jax.ShapeDtypeStruct((_N, _D), jnp.float32),
    )(x, a0, a1, d0, d1, w_upd)


_sc_mesh = plsc.VectorSubcoreMesh(core_axis_name="c", subcore_axis_name="s")


@functools.partial(
    pl.kernel,
    out_type=[jax.ShapeDtypeStruct((_NC * _NPAD, _D), jnp.float32),
              jax.ShapeDtypeStruct((_NC * _NPAD,), jnp.float32)],
    mesh=_sc_mesh,
    compiler_params=pltpu.CompilerParams(
        needs_layout_passes=False, use_tc_tiling_on_sc=False),
    scratch_types=[
        pltpu.VMEM_SHARED((_NPAD, _D), jnp.float32),
        pltpu.VMEM_SHARED((_NPAD,), jnp.float32),
        pltpu.VMEM((_T, _D), jnp.float32),
        pltpu.VMEM((_CB,), jnp.int32),
        pltpu.VMEM((_CB,), jnp.int32),
        pltpu.VMEM((_CB,), jnp.int32),
        pltpu.VMEM((_CB,), jnp.float32),
        pltpu.VMEM((_CB,), jnp.float32),
        pltpu.VMEM((_NSUB, _SUB), jnp.int32),
        pltpu.VMEM((_NSUB, _SUB), jnp.int32),
        pltpu.VMEM((_NSUB, _SUB), jnp.int32),
        pltpu.VMEM((_SUB, _D), jnp.float32),
        pltpu.VMEM((_SUB, _D), jnp.float32),
        pltpu.SemaphoreType.DMA,
        pltpu.SemaphoreType.DMA,
        pltpu.SemaphoreType.DMA,
    ],
)
def _sc_edges(y_hbm, sflat_hbm, p_hbm, src_hbm, dst_hbm, et_hbm, attn_hbm,
              z2d_hbm, zd_hbm, out_hbm, dnout_hbm,
              aggr_sh, denom_sh, p_l,
              src_b, dst_b, et_b, na_b, sv_b,
              sidx2, yidx2, dst2, rows0, rows1, semg, sems, semd):
    c = lax.axis_index("c")
    s = lax.axis_index("s")
    base = c * _EPC + s * _EPT
    col0 = lax.iota(jnp.int32, 16)

    pltpu.sync_copy(z2d_hbm, aggr_sh.at[pl.ds(s * _RPT, _RPT)])
    pltpu.sync_copy(zd_hbm, denom_sh.at[pl.ds(s * _RPT, _RPT)])
    pltpu.sync_copy(p_hbm, p_l)
    plsc.subcore_barrier()

    def _compute(rbuf, r):
        @plsc.parallel_loop(0, _SUB, 1, unroll=2)
        def _edge(e):
            psplat = jnp.full((16,), r * _SUB + e, jnp.int32)
            rsplat = plsc.load_gather(et_b, [psplat])
            na = plsc.load_gather(na_b, [psplat])
            for q in range(_D // 16):
                pv = plsc.load_gather(p_l, [rsplat, col0 + q * 16])
                yv = rbuf[e, pl.ds(q * 16, 16)]
                u = jnp.exp(yv + pv)
                rbuf[e, pl.ds(q * 16, 16)] = (1.0 - 2.0 / (u + 1.0)) * na

    def _wait_gather(rbuf):
        pltpu.make_async_copy(y_hbm.at[yidx2.at[0]], rbuf, semg).wait()

    def _drain_scatter(rbuf):
        pltpu.make_async_copy(rbuf, aggr_sh.at[dst2.at[0]], sems).wait()

    def chunk(bc, carry):
        off = base + bc * _CB
        pltpu.sync_copy(src_hbm.at[pl.ds(off, _CB)], src_b)
        pltpu.sync_copy(dst_hbm.at[pl.ds(off, _CB)], dst_b)
        pltpu.sync_copy(et_hbm.at[pl.ds(off, _CB)], et_b)
        pltpu.sync_copy(attn_hbm.at[pl.ds(off, _CB)], na_b)

        @plsc.parallel_loop(0, _NSUB, 1, unroll=2)
        def _mk(r):
            for q in range(_VPS):
                j = r * _SUB + q * 16
                sv = src_b[pl.ds(j, 16)]
                ev = et_b[pl.ds(j, 16)]
                sidx2[r, pl.ds(q * 16, 16)] = sv * _T + ev
                yidx2[r, pl.ds(q * 16, 16)] = sv
                dst2[r, pl.ds(q * 16, 16)] = dst_b[pl.ds(j, 16)]

        def fire_g(r, _):
            pltpu.async_copy(sflat_hbm.at[sidx2.at[r]],
                             sv_b.at[pl.ds(r * _SUB, _SUB)], semg)
            return _
        lax.fori_loop(0, _NSUB, fire_g, None)

        def drain_g(r, _):
            pltpu.make_async_copy(sflat_hbm.at[sidx2.at[0]],
                                  sv_b.at[pl.ds(0, _SUB)], semg).wait()
            return _
        lax.fori_loop(0, _NSUB, drain_g, None)

        @plsc.parallel_loop(0, _CB // 16, 1, unroll=4)
        def _sig(j):
            sv = sv_b[pl.ds(j * 16, 16)]
            t = 1.0 / (1.0 + jnp.exp(sv))
            na_b[pl.ds(j * 16, 16)] = na_b[pl.ds(j * 16, 16)] * t

        def fire_d(r, _):
            pltpu.async_copy(na_b.at[pl.ds(r * _SUB, _SUB)],
                             denom_sh.at[dst2.at[r]], semd, add=True)
            return _
        lax.fori_loop(0, _NSUB, fire_d, None)

        pltpu.async_copy(y_hbm.at[yidx2.at[0]], rows0, semg)

        def pair(k, _):
            r0 = 2 * k
            _wait_gather(rows0)

            @pl.when(k > 0)
            def _():
                _drain_scatter(rows1)
            pltpu.async_copy(y_hbm.at[yidx2.at[r0 + 1]], rows1, semg)
            _compute(rows0, r0)
            pltpu.async_copy(rows0, aggr_sh.at[dst2.at[r0]], sems, add=True)
            _wait_gather(rows1)
            _drain_scatter(rows0)
            pltpu.async_copy(y_hbm.at[yidx2.at[r0 + 2]], rows0, semg)
            _compute(rows1, r0 + 1)
            pltpu.async_copy(rows1, aggr_sh.at[dst2.at[r0 + 1]], sems, add=True)
            return _
        lax.fori_loop(0, _NSUB // 2, pair, None)
        _wait_gather(rows0)
        _drain_scatter(rows1)
        _compute(rows0, _NSUB - 1)
        pltpu.async_copy(rows0, aggr_sh.at[dst2.at[_NSUB - 1]], sems, add=True)
        _drain_scatter(rows0)

        def drain_d(r, _):
            pltpu.make_async_copy(na_b.at[pl.ds(0, _SUB)],
                                  denom_sh.at[dst2.at[0]], semd).wait()
            return _
        lax.fori_loop(0, _NSUB, drain_d, None)
        return carry
    lax.fori_loop(0, _NBC, chunk, None)

    plsc.subcore_barrier()
    pltpu.sync_copy(aggr_sh.at[pl.ds(s * _RPT, _RPT)],
                    out_hbm.at[pl.ds(c * _NPAD + s * _RPT, _RPT)])
    pltpu.sync_copy(denom_sh.at[pl.ds(s * _RPT, _RPT)],
                    dnout_hbm.at[pl.ds(c * _NPAD + s * _RPT, _RPT)])


def kernel(x, attn, W_msg, etype_params, W_upd, edge_index, edge_type):
    src = edge_index[0]
    dst = edge_index[1]
    y, S = _pre(x, W_msg, etype_params)
    sflat = S.reshape(_N * _T)
    pT = etype_params * 2.0
    z2d = jnp.zeros((_RPT, _D), jnp.float32)
    zd = jnp.zeros((_RPT,), jnp.float32)
    aggr_flat, dn_flat = _sc_edges(y, sflat, pT, src, dst, edge_type, attn,
                                   z2d, zd)
    a = aggr_flat.reshape(_NC, _NPAD, _D)
    dn = dn_flat.reshape(_NC, _NPAD)
    return _post(x, a[0, :_N], a[1, :_N],
                 dn[0, :_N].reshape(_N, 1), dn[1, :_N].reshape(_N, 1), W_upd)


# speedup vs baseline: 1.6110x; 1.0203x over previous
import functools

import jax
import jax.numpy as jnp
import numpy as np
from jax import lax
from jax.experimental import pallas as pl
from jax.experimental.pallas import tpu as pltpu
from jax.experimental.pallas import tpu_sc as plsc

_N = 10000
_E = 320000
_D = 128
_T = 16
_NC = 2
_NS = 16
_NPAD = 10000
_RPT = _NPAD // _NS
_NPD = 10240
_RPD = _NPD // _NS
_EPC = _E // _NC
_EPT = _EPC // _NS
_CB = 2000
_NBC = _EPT // _CB
_SUB = 80
_NSUB = _CB // _SUB
_VPS = _SUB // 16


def _pre_body(x_ref, w_ref, p_ref, y_ref, s_ref):
    xb = x_ref[...]
    y_ref[...] = jnp.dot(xb, w_ref[...], preferred_element_type=jnp.float32) * 2.0
    s_ref[...] = jnp.dot(xb, p_ref[...].T, preferred_element_type=jnp.float32) * (
        -1.0 / np.sqrt(np.float32(_D)))


def _pre(x, w_msg, etype_params):
    bn = 1000
    grid = (_N // bn,)
    return pl.pallas_call(
        _pre_body,
        grid=grid,
        in_specs=[
            pl.BlockSpec((bn, _D), lambda i: (i, 0)),
            pl.BlockSpec((_D, _D), lambda i: (0, 0)),
            pl.BlockSpec((_T, _D), lambda i: (0, 0)),
        ],
        out_specs=[
            pl.BlockSpec((bn, _D), lambda i: (i, 0)),
            pl.BlockSpec((bn, _T), lambda i: (i, 0)),
        ],
        out_shape=[
            jax.ShapeDtypeStruct((_N, _D), jnp.float32),
            jax.ShapeDtypeStruct((_N, _T), jnp.float32),
        ],
    )(x, w_msg, etype_params)


def _post_body(x_ref, a0_ref, a1_ref, d0_ref, d1_ref, w_ref, o_ref):
    scale = 1.0 / (d0_ref[...] + d1_ref[...] + 1e-9)
    aggr = (a0_ref[...] + a1_ref[...]) * scale
    o_ref[...] = x_ref[...] + jnp.tanh(
        jnp.dot(aggr, w_ref[...], preferred_element_type=jnp.float32))


def _post(x, aggr2, d0, d1, w_upd):
    bn = 1000
    nb = _N // bn
    grid = (nb,)
    return pl.pallas_call(
        _post_body,
        grid=grid,
        in_specs=[
            pl.BlockSpec((bn, _D), lambda i: (i, 0)),
            pl.BlockSpec((bn, _D), lambda i: (i, 0)),
            pl.BlockSpec((bn, _D), lambda i: (i + _N // bn, 0)),
            pl.BlockSpec((bn, 1), lambda i: (i, 0)),
            pl.BlockSpec((bn, 1), lambda i: (i, 0)),
            pl.BlockSpec((_D, _D), lambda i: (0, 0)),
        ],
        out_specs=pl.BlockSpec((bn, _D), lambda i: (i, 0)),
        out_shape=jax.ShapeDtypeStruct((_N, _D), jnp.float32),
    )(x, aggr2, aggr2, d0, d1, w_upd)


_sc_mesh = plsc.VectorSubcoreMesh(core_axis_name="c", subcore_axis_name="s")


@functools.partial(
    pl.kernel,
    out_type=[jax.ShapeDtypeStruct((_NC * _NPAD, _D), jnp.float32),
              jax.ShapeDtypeStruct((_NC * _NPD,), jnp.float32)],
    mesh=_sc_mesh,
    compiler_params=pltpu.CompilerParams(
        needs_layout_passes=False, use_tc_tiling_on_sc=False),
    scratch_types=[
        pltpu.VMEM_SHARED((_NPAD, _D), jnp.float32),
        pltpu.VMEM_SHARED((_NPD,), jnp.float32),
        pltpu.VMEM((_T, _D), jnp.float32),
        pltpu.VMEM((_CB,), jnp.int32),
        pltpu.VMEM((_CB,), jnp.int32),
        pltpu.VMEM((_CB,), jnp.int32),
        pltpu.VMEM((_CB,), jnp.float32),
        pltpu.VMEM((_CB,), jnp.float32),
        pltpu.VMEM((_NSUB, _SUB), jnp.int32),
        pltpu.VMEM((_NSUB, _SUB), jnp.int32),
        pltpu.VMEM((_NSUB, _SUB), jnp.int32),
        pltpu.VMEM((_SUB, _D), jnp.float32),
        pltpu.VMEM((_SUB, _D), jnp.float32),
        pltpu.SemaphoreType.DMA,
        pltpu.SemaphoreType.DMA,
        pltpu.SemaphoreType.DMA,
    ],
)
def _sc_edges(y_hbm, sflat_hbm, p_hbm, src_hbm, dst_hbm, et_hbm, attn_hbm,
              z2d_hbm, zd_hbm, out_hbm, dnout_hbm,
              aggr_sh, denom_sh, p_l,
              src_b, dst_b, et_b, na_b, sv_b,
              sidx2, yidx2, dst2, rows0, rows1, semg, sems, semd):
    c = lax.axis_index("c")
    s = lax.axis_index("s")
    base = c * _EPC + s * _EPT
    col0 = lax.iota(jnp.int32, 16)

    pltpu.sync_copy(z2d_hbm, aggr_sh.at[pl.ds(s * _RPT, _RPT)])
    pltpu.sync_copy(zd_hbm, denom_sh.at[pl.ds(s * _RPD, _RPD)])
    pltpu.sync_copy(p_hbm, p_l)
    plsc.subcore_barrier()

    def _compute(rbuf, r):
        @plsc.parallel_loop(0, _SUB, 1, unroll=2)
        def _edge(e):
            psplat = jnp.full((16,), r * _SUB + e, jnp.int32)
            rsplat = plsc.load_gather(et_b, [psplat])
            na = plsc.load_gather(na_b, [psplat])
            for q in range(_D // 16):
                pv = plsc.load_gather(p_l, [rsplat, col0 + q * 16])
                yv = rbuf[e, pl.ds(q * 16, 16)]
                u = jnp.exp(yv + pv)
                rbuf[e, pl.ds(q * 16, 16)] = (1.0 - 2.0 / (u + 1.0)) * na

    def _wait_gather(rbuf):
        pltpu.make_async_copy(y_hbm.at[yidx2.at[0]], rbuf, semg).wait()

    def _drain_scatter(rbuf):
        pltpu.make_async_copy(rbuf, aggr_sh.at[dst2.at[0]], sems).wait()

    def chunk(bc, carry):
        off = base + bc * _CB
        pltpu.sync_copy(src_hbm.at[pl.ds(off, _CB)], src_b)
        pltpu.sync_copy(dst_hbm.at[pl.ds(off, _CB)], dst_b)
        pltpu.sync_copy(et_hbm.at[pl.ds(off, _CB)], et_b)
        pltpu.sync_copy(attn_hbm.at[pl.ds(off, _CB)], na_b)

        @plsc.parallel_loop(0, _NSUB, 1, unroll=2)
        def _mk(r):
            for q in range(_VPS):
                j = r * _SUB + q * 16
                sv = src_b[pl.ds(j, 16)]
                ev = et_b[pl.ds(j, 16)]
                sidx2[r, pl.ds(q * 16, 16)] = sv * _T + ev
                yidx2[r, pl.ds(q * 16, 16)] = sv
                dst2[r, pl.ds(q * 16, 16)] = dst_b[pl.ds(j, 16)]

        def fire_g(r, _):
            pltpu.async_copy(sflat_hbm.at[sidx2.at[r]],
                             sv_b.at[pl.ds(r * _SUB, _SUB)], semg)
            return _
        lax.fori_loop(0, _NSUB, fire_g, None)

        def drain_g(r, _):
            pltpu.make_async_copy(sflat_hbm.at[sidx2.at[0]],
                                  sv_b.at[pl.ds(0, _SUB)], semg).wait()
            return _
        lax.fori_loop(0, _NSUB, drain_g, None)

        @plsc.parallel_loop(0, _CB // 16, 1, unroll=4)
        def _sig(j):
            sv = sv_b[pl.ds(j * 16, 16)]
            t = 1.0 / (1.0 + jnp.exp(sv))
            na_b[pl.ds(j * 16, 16)] = na_b[pl.ds(j * 16, 16)] * t

        def fire_d(r, _):
            pltpu.async_copy(na_b.at[pl.ds(r * _SUB, _SUB)],
                             denom_sh.at[dst2.at[r]], semd, add=True)
            return _
        lax.fori_loop(0, _NSUB, fire_d, None)

        pltpu.async_copy(y_hbm.at[yidx2.at[0]], rows0, semg)

        def pair(k, _):
            r0 = 2 * k
            _wait_gather(rows0)

            @pl.when(k > 0)
            def _():
                _drain_scatter(rows1)
            pltpu.async_copy(y_hbm.at[yidx2.at[r0 + 1]], rows1, semg)
            _compute(rows0, r0)
            pltpu.async_copy(rows0, aggr_sh.at[dst2.at[r0]], sems, add=True)
            _wait_gather(rows1)
            _drain_scatter(rows0)
            pltpu.async_copy(y_hbm.at[yidx2.at[r0 + 2]], rows0, semg)
            _compute(rows1, r0 + 1)
            pltpu.async_copy(rows1, aggr_sh.at[dst2.at[r0 + 1]], sems, add=True)
            return _
        lax.fori_loop(0, _NSUB // 2, pair, None)
        _wait_gather(rows0)
        _drain_scatter(rows1)
        _compute(rows0, _NSUB - 1)
        pltpu.async_copy(rows0, aggr_sh.at[dst2.at[_NSUB - 1]], sems, add=True)
        _drain_scatter(rows0)

        def drain_d(r, _):
            pltpu.make_async_copy(na_b.at[pl.ds(0, _SUB)],
                                  denom_sh.at[dst2.at[0]], semd).wait()
            return _
        lax.fori_loop(0, _NSUB, drain_d, None)
        return carry
    lax.fori_loop(0, _NBC, chunk, None)

    plsc.subcore_barrier()
    pltpu.sync_copy(aggr_sh.at[pl.ds(s * _RPT, _RPT)],
                    out_hbm.at[pl.ds(c * _NPAD + s * _RPT, _RPT)])
    pltpu.sync_copy(denom_sh.at[pl.ds(s * _RPD, _RPD)],
                    dnout_hbm.at[pl.ds(c * _NPD + s * _RPD, _RPD)])


def kernel(x, attn, W_msg, etype_params, W_upd, edge_index, edge_type):
    src = edge_index[0]
    dst = edge_index[1]
    y, S = _pre(x, W_msg, etype_params)
    sflat = S.reshape(_N * _T)
    pT = etype_params * 2.0
    z2d = jnp.zeros((_RPT, _D), jnp.float32)
    zd = jnp.zeros((_RPD,), jnp.float32)
    aggr_flat, dn_flat = _sc_edges(y, sflat, pT, src, dst, edge_type, attn,
                                   z2d, zd)
    dn = dn_flat.reshape(_NC, _NPD)
    return _post(x, aggr_flat, dn[0, :_N].reshape(_N, 1),
                 dn[1, :_N].reshape(_N, 1), W_upd)
